# baseline (speedup 1.0000x reference)
import jax
import jax.numpy as jnp
from jax.experimental import pallas as pl

IN_C = 128
HID = 128
B = 64
NCOV = 8


def _gcn(x, edge_index, W, b):
    n = x.shape[0]
    xw = x @ W.T
    loop = jnp.arange(n, dtype=edge_index.dtype)
    row = jnp.concatenate([edge_index[0], loop])
    col = jnp.concatenate([edge_index[1], loop])
    ew = jnp.concatenate([jnp.ones((edge_index.shape[1],), dtype=x.dtype), 2.0 * jnp.ones((n,), dtype=x.dtype)])
    deg = jnp.zeros((n,), dtype=x.dtype).at[col].add(ew)
    dinv = jnp.where(deg > 0, deg ** -0.5, 0.0)
    norm = dinv[row] * ew * dinv[col]
    out = jnp.zeros_like(xw).at[col].add(norm[:, None] * xw[row])
    return out + b


def _enc(x, edge_index, lin_w, lin_b, c1_w, c1_b, c2_w, c2_b):
    out = x @ lin_w.T + lin_b
    id1 = out
    out1 = jax.nn.relu(_gcn(out, edge_index, c1_w, c1_b)) + id1
    out2 = _gcn(out1, edge_index, c2_w, c2_b) + out1
    return out2


def _pool(x, batch, nb):
    s = jax.ops.segment_sum(x, batch, num_segments=nb)
    cnt = jax.ops.segment_sum(jnp.ones((x.shape[0],), dtype=x.dtype), batch, num_segments=nb)
    return s / jnp.clip(cnt, 1.0)[:, None]


def _identity_pallas(x):
    def body(x_ref, o_ref):
        o_ref[...] = x_ref[...]
    return pl.pallas_call(body, out_shape=jax.ShapeDtypeStruct(x.shape, x.dtype))(x)


def kernel(x, edge_index, cov, batch, lin_w, lin_b, c1_w, c1_b, c2_w, c2_b, w_ih, w_hh, b_ih, b_hh, w0_w, w0_b, attn_w, attn_b, l1_w, l1_b, l2_w, l2_b, h0, c0):
    x1 = x[:, :IN_C]
    x2 = x[:, IN_C:2 * IN_C]
    x3 = x[:, 2 * IN_C:]
    enc = lambda xi: _enc(xi, edge_index, lin_w, lin_b, c1_w, c1_b, c2_w, c2_b)
    o1 = _pool(enc(x1), batch, B)
    o2 = _pool(enc(x2), batch, B)
    o3 = _pool(enc(x3), batch, B)
    seq = jnp.stack([o1, o2, o3], axis=1)

    def step(carry, xt):
        h, c = carry
        g = xt @ w_ih.T + b_ih + h @ w_hh.T + b_hh
        i, f, gg, o = jnp.split(g, 4, axis=-1)
        i = jax.nn.sigmoid(i); f = jax.nn.sigmoid(f)
        gg = jnp.tanh(gg); o = jax.nn.sigmoid(o)
        c = f * c + i * gg
        h = o * jnp.tanh(c)
        return (h, c), h

    (hT, cT), outs = jax.lax.scan(step, (h0[0], c0[0]), jnp.swapaxes(seq, 0, 1))
    rnn_out = jnp.swapaxes(outs, 0, 1)
    rnn_out = jnp.tanh(rnn_out @ w0_w.T + w0_b)
    last = jnp.tile(hT[:, None, :], (1, rnn_out.shape[1], 1))
    merged = jnp.concatenate([rnn_out, last], axis=2)
    w = (merged @ attn_w.T + attn_b)[:, :, 0]
    attn = jax.nn.softmax(w, axis=1)
    out = jnp.einsum('btd,bt->bd', rnn_out, attn)
    out = jax.nn.relu(out @ l1_w.T + l1_b)
    out = jnp.concatenate([out, cov.reshape(-1, NCOV)], axis=1)
    out = out @ l2_w.T + l2_b
    out = _identity_pallas(out)
    return (attn, out)


# speedup vs baseline: 8.8526x; 8.8526x over previous
import functools

import jax
import jax.numpy as jnp
from jax import lax
from jax.experimental import pallas as pl
from jax.experimental.pallas import tpu as pltpu
from jax.experimental.pallas import tpu_sc as plsc

N = 10000
NPAD = 10240
E = 320000
B = 64
HID = 128
NCOV = 8
NC, NS, L = 2, 16, 16
NW = NC * NS
EPW = E // NW
CH = 80
NCHUNK = EPW // CH
STRIPE = NPAD // NS
RB = 1280
GR = NPAD // RB


def _sc_mesh():
    return plsc.VectorSubcoreMesh(core_axis_name="c", subcore_axis_name="s",
                                  num_cores=NC, num_subcores=NS)


@functools.cache
def _deg_kernel():
    return functools.partial(
        pl.kernel,
        out_type=jax.ShapeDtypeStruct((NW, NPAD), jnp.float32),
        mesh=_sc_mesh(),
        compiler_params=pltpu.CompilerParams(needs_layout_passes=False),
        scratch_types=[
            pltpu.VMEM((EPW,), jnp.int32),
            pltpu.VMEM((NPAD,), jnp.float32),
        ],
    )(_deg_body)


def _deg_call(col):
    return _deg_kernel()(col)


def _deg_body(col_hbm, out_hbm, colv, acc):
    cid = lax.axis_index("c")
    sid = lax.axis_index("s")
    wid = cid * NS + sid
    pltpu.sync_copy(col_hbm.at[pl.ds(wid * EPW, EPW)], colv)
    zeros = jnp.zeros((L,), jnp.float32)
    ones = jnp.ones((L,), jnp.float32)

    def zbody(i, _):
        acc[pl.ds(i * L, L)] = zeros
        return _

    lax.fori_loop(0, NPAD // L, zbody, None)

    def hbody(i, _):
        idx = colv[pl.ds(i * L, L)]
        plsc.addupdate_scatter(acc, [idx], ones)
        return _

    lax.fori_loop(0, EPW // L, hbody, None)
    pltpu.sync_copy(acc, out_hbm.at[wid])


@functools.cache
def _scatter_kernel():
    return functools.partial(
        pl.kernel,
        out_type=jax.ShapeDtypeStruct((NC * 3 * NPAD, HID), jnp.float32),
        mesh=_sc_mesh(),
        compiler_params=pltpu.CompilerParams(needs_layout_passes=False),
        scratch_types=[
            pltpu.VMEM((CH,), jnp.int32),
            pltpu.VMEM((CH,), jnp.int32),
            pltpu.VMEM((CH, HID), jnp.float32),
            pltpu.VMEM((128, HID), jnp.float32),
            pltpu.VMEM_SHARED((NPAD, HID), jnp.float32),
            pltpu.SemaphoreType.DMA,
        ],
    )(_scatter_body)


def _scatter_call(row3, col, z):
    return _scatter_kernel()(row3, col, z)


def _scatter_body(row3_hbm, col_hbm, z_hbm, out_hbm,
                  rowv, colv, gbuf, zbuf, acc_sh, sem):
    cid = lax.axis_index("c")
    sid = lax.axis_index("s")
    wid = cid * NS + sid
    ebase = wid * EPW
    zeros = jnp.zeros((L,), jnp.float32)

    def zb(t, _):
        zbuf[t // 8, pl.ds((t % 8) * L, L)] = zeros
        return _

    lax.fori_loop(0, 128 * HID // L, zb, None)

    for s in range(3):
        for k in range(STRIPE // 128):
            pltpu.sync_copy(zbuf, acc_sh.at[pl.ds(sid * STRIPE + k * 128, 128)])
        plsc.subcore_barrier()

        def chunk(i, _):
            pltpu.sync_copy(row3_hbm.at[pl.ds(s * E + ebase + i * CH, CH)], rowv)
            pltpu.sync_copy(col_hbm.at[pl.ds(ebase + i * CH, CH)], colv)
            pltpu.async_copy(z_hbm.at[rowv], gbuf, sem).wait()
            pltpu.sync_copy(gbuf, acc_sh.at[colv], add=True)
            return _

        lax.fori_loop(0, NCHUNK, chunk, None)
        plsc.subcore_barrier()
        obase = (cid * 3 + s) * NPAD + sid * STRIPE
        pltpu.sync_copy(acc_sh.at[pl.ds(sid * STRIPE, STRIPE)],
                        out_hbm.at[pl.ds(obase, STRIPE)])


def _nt(a, b):
    return lax.dot_general(a, b, (((1,), (1,)), ((), ())),
                           preferred_element_type=jnp.float32)


def _ka_body(x_ref, degp_ref, lin_w_ref, lin_b_ref, c1_w_ref,
             h_ref, z1_ref, dinv_ref):
    deg = jnp.sum(degp_ref[...], axis=0, keepdims=True) + 2.0
    dlane = lax.rsqrt(deg)
    ones = jnp.ones((1, HID), jnp.float32)
    dinv = lax.dot_general(dlane, ones, (((0,), (0,)), ((), ())),
                           preferred_element_type=jnp.float32)
    dinv_ref[...] = dinv
    for s in range(3):
        xs = x_ref[:, s * HID:(s + 1) * HID]
        hs = _nt(xs, lin_w_ref[...]) + lin_b_ref[...]
        xw = _nt(hs, c1_w_ref[...])
        h_ref[s] = hs
        z1_ref[s] = dinv * xw


def _kb_body(dinv_ref, h_ref, z1_ref, p_ref, c1_b_ref, c2_w_ref,
             out1_ref, z2_ref):
    dinv = dinv_ref[...]
    agg = p_ref[0, 0] + p_ref[1, 0]
    conv1 = dinv * agg + 2.0 * dinv * z1_ref[0] + c1_b_ref[...]
    o1 = jax.nn.relu(conv1) + h_ref[0]
    out1_ref[0] = o1
    z2_ref[0] = dinv * _nt(o1, c2_w_ref[...])


def _kc_body(dinv_ref, out1_ref, z2_ref, p_ref, c2_b_ref, batch_ref,
             seq_ref, pooled, cnt):
    i = pl.program_id(1)
    dinv = dinv_ref[...]
    agg = p_ref[0, 0] + p_ref[1, 0]
    o2 = dinv * agg + 2.0 * dinv * z2_ref[0] + c2_b_ref[...] + out1_ref[0]
    bt = batch_ref[...]
    ohT = (jnp.broadcast_to(bt, (B, RB))
           == lax.broadcasted_iota(jnp.int32, (B, RB), 0)).astype(jnp.float32)

    @pl.when(i == 0)
    def _():
        pooled[...] = jnp.zeros_like(pooled)
        cnt[...] = jnp.zeros_like(cnt)

    pooled[...] += lax.dot_general(ohT, o2, (((1,), (0,)), ((), ())),
                                   preferred_element_type=jnp.float32)
    cnt[...] += jnp.sum(ohT, axis=1, keepdims=True)

    @pl.when(i == GR - 1)
    def _():
        seq_ref[0] = pooled[...] / jnp.maximum(cnt[...], 1.0)


def _kd_body(seq_ref, cov_ref, w_ih_ref, w_hh_ref, b_ih_ref, b_hh_ref,
             w0_w_ref, w0_b_ref, aw1_ref, aw2_ref, ab_ref,
             l1_w_ref, l1_b_ref, l2a_ref, l2b_ref, l2_b_ref,
             h0_ref, c0_ref, attn_ref, out_ref):
    h = h0_ref[...]
    c = c0_ref[...]
    hs = []
    for t in range(3):
        xt = seq_ref[t]
        g = (_nt(xt, w_ih_ref[...]) + b_ih_ref[...]
             + _nt(h, w_hh_ref[...]) + b_hh_ref[...])
        ii = jax.nn.sigmoid(g[:, 0 * HID:1 * HID])
        ff = jax.nn.sigmoid(g[:, 1 * HID:2 * HID])
        gg = jnp.tanh(g[:, 2 * HID:3 * HID])
        oo = jax.nn.sigmoid(g[:, 3 * HID:4 * HID])
        c = ff * c + ii * gg
        h = oo * jnp.tanh(c)
        hs.append(h)
    hT = hs[-1]
    aw1 = aw1_ref[...]
    aw2 = aw2_ref[...]
    ab = ab_ref[0, 0]
    rs, ws = [], []
    for t in range(3):
        rt = jnp.tanh(_nt(hs[t], w0_w_ref[...]) + w0_b_ref[...])
        wt = (jnp.sum(rt * aw1, axis=1, keepdims=True)
              + jnp.sum(hT * aw2, axis=1, keepdims=True) + ab)
        rs.append(rt)
        ws.append(wt)
    m = jnp.maximum(ws[0], jnp.maximum(ws[1], ws[2]))
    es = [jnp.exp(w - m) for w in ws]
    tot = es[0] + es[1] + es[2]
    als = [e / tot for e in es]
    feat = als[0] * rs[0] + als[1] * rs[1] + als[2] * rs[2]
    l1o = jax.nn.relu(_nt(feat, l1_w_ref[...]) + l1_b_ref[...])
    out = (_nt(l1o, l2a_ref[...]) + _nt(cov_ref[...], l2b_ref[...])
           + l2_b_ref[...])
    attn_ref[...] = jnp.concatenate(
        [als[0], als[1], als[2], jnp.zeros((B, HID - 3), jnp.float32)], axis=1)
    out_ref[...] = jnp.concatenate(
        [out, jnp.zeros((B, HID - 2), jnp.float32)], axis=1)


def _full(shape):
    return pl.BlockSpec(shape, lambda *_: tuple(0 for _ in shape))


_ka = pl.pallas_call(
    _ka_body,
    grid=(GR,),
    in_specs=[
        pl.BlockSpec((RB, 3 * HID), lambda i: (i, 0)),
        pl.BlockSpec((NW, RB), lambda i: (0, i)),
        _full((HID, HID)),
        _full((1, HID)),
        _full((HID, HID)),
    ],
    out_specs=[
        pl.BlockSpec((3, RB, HID), lambda i: (0, i, 0)),
        pl.BlockSpec((3, RB, HID), lambda i: (0, i, 0)),
        pl.BlockSpec((RB, HID), lambda i: (i, 0)),
    ],
    out_shape=[
        jax.ShapeDtypeStruct((3, NPAD, HID), jnp.float32),
        jax.ShapeDtypeStruct((3, NPAD, HID), jnp.float32),
        jax.ShapeDtypeStruct((NPAD, HID), jnp.float32),
    ],
)

_kb = pl.pallas_call(
    _kb_body,
    grid=(3, GR),
    in_specs=[
        pl.BlockSpec((RB, HID), lambda s, i: (i, 0)),
        pl.BlockSpec((1, RB, HID), lambda s, i: (s, i, 0)),
        pl.BlockSpec((1, RB, HID), lambda s, i: (s, i, 0)),
        pl.BlockSpec((NC, 1, RB, HID), lambda s, i: (0, s, i, 0)),
        _full((1, HID)),
        _full((HID, HID)),
    ],
    out_specs=[
        pl.BlockSpec((1, RB, HID), lambda s, i: (s, i, 0)),
        pl.BlockSpec((1, RB, HID), lambda s, i: (s, i, 0)),
    ],
    out_shape=[
        jax.ShapeDtypeStruct((3, NPAD, HID), jnp.float32),
        jax.ShapeDtypeStruct((3, NPAD, HID), jnp.float32),
    ],
)

_kc = pl.pallas_call(
    _kc_body,
    grid=(3, GR),
    in_specs=[
        pl.BlockSpec((RB, HID), lambda s, i: (i, 0)),
        pl.BlockSpec((1, RB, HID), lambda s, i: (s, i, 0)),
        pl.BlockSpec((1, RB, HID), lambda s, i: (s, i, 0)),
        pl.BlockSpec((NC, 1, RB, HID), lambda s, i: (0, s, i, 0)),
        _full((1, HID)),
        pl.BlockSpec((1, RB), lambda s, i: (0, i)),
    ],
    out_specs=[pl.BlockSpec((1, B, HID), lambda s, i: (s, 0, 0))],
    out_shape=[jax.ShapeDtypeStruct((3, B, HID), jnp.float32)],
    scratch_shapes=[
        pltpu.VMEM((B, HID), jnp.float32),
        pltpu.VMEM((B, 1), jnp.float32),
    ],
)

_kd = pl.pallas_call(
    _kd_body,
    out_shape=[
        jax.ShapeDtypeStruct((B, HID), jnp.float32),
        jax.ShapeDtypeStruct((B, HID), jnp.float32),
    ],
)


def kernel(x, edge_index, cov, batch, lin_w, lin_b, c1_w, c1_b, c2_w, c2_b,
           w_ih, w_hh, b_ih, b_hh, w0_w, w0_b, attn_w, attn_b,
           l1_w, l1_b, l2_w, l2_b, h0, c0):
    f32 = jnp.float32
    x_pad = jnp.pad(x, ((0, NPAD - N), (0, 0)))
    batch_pad = jnp.pad(batch, (0, NPAD - N),
                        constant_values=B).reshape(1, NPAD)
    row = edge_index[0]
    col = edge_index[1]
    row3 = (row[None, :]
            + (jnp.arange(3, dtype=jnp.int32) * NPAD)[:, None]).reshape(-1)

    degp = _deg_call(col)
    enc_h, z1, dinv_b = _ka(x_pad, degp, lin_w, lin_b.reshape(1, HID), c1_w)
    p1 = _scatter_call(row3, col, z1.reshape(3 * NPAD, HID))
    p1 = p1.reshape(NC, 3, NPAD, HID)
    out1, z2 = _kb(dinv_b, enc_h, z1, p1, c1_b.reshape(1, HID), c2_w)
    p2 = _scatter_call(row3, col, z2.reshape(3 * NPAD, HID))
    p2 = p2.reshape(NC, 3, NPAD, HID)
    (seq,) = _kc(dinv_b, out1, z2, p2, c2_b.reshape(1, HID), batch_pad)

    attn_p, out_p = _kd(
        seq, cov.astype(f32), w_ih, w_hh,
        b_ih.reshape(1, 4 * HID), b_hh.reshape(1, 4 * HID),
        w0_w, w0_b.reshape(1, HID),
        attn_w[:, :HID], attn_w[:, HID:], attn_b.reshape(1, 1),
        l1_w, l1_b.reshape(1, 8),
        l2_w[:, :8], l2_w[:, 8:], l2_b.reshape(1, 2),
        h0[0], c0[0])
    return (attn_p[:, :3], out_p[:, :2])
